# baseline (speedup 1.0000x reference)
import functools

import jax
import jax.numpy as jnp
from jax import lax
from jax.experimental import pallas as pl
from jax.experimental.pallas import tpu as pltpu
from jax.experimental.pallas import tpu_sc as plsc

_NC = 2
_NS = 16
_NW = _NC * _NS
_CHUNK = 512
_G = 1


def _gather_body(n_groups, d, table_hbm, idx_hbm, out_hbm,
                 idx_v, rows0, rows1, gsem0, gsem1, ssem0, ssem1):
    wid = lax.axis_index("s") * _NC + lax.axis_index("c")
    group_rows = _G * _CHUNK
    base = wid * (n_groups * group_rows)
    rows = (rows0, rows1)
    gsem = (gsem0, gsem1)
    ssem = (ssem0, ssem1)
    pltpu.sync_copy(idx_hbm.at[wid], idx_v)

    def pair(t, carry):
        for b in range(2):
            g = 2 * t + b
            @pl.when(t >= 1)
            def _():
                pltpu.make_async_copy(
                    rows[b], out_hbm.at[pl.ds(0, group_rows)], ssem[b]).wait()

            handles = [
                pltpu.async_copy(
                    table_hbm.at[idx_v.at[g * _G + u]],
                    rows[b].at[pl.ds(u * _CHUNK, _CHUNK)], gsem[b])
                for u in range(_G)
            ]
            for h in handles:
                h.wait()
            pltpu.async_copy(
                rows[b], out_hbm.at[pl.ds(base + g * group_rows, group_rows)],
                ssem[b])
        return carry

    lax.fori_loop(0, n_groups // 2, pair, 0)
    for b in range(2):
        pltpu.make_async_copy(
            rows[b], out_hbm.at[pl.ds(0, group_rows)], ssem[b]).wait()


@functools.partial(jax.jit, static_argnames=("n_groups", "d"))
def _gather(table, idx, n_groups, d):
    mesh = plsc.VectorSubcoreMesh(core_axis_name="c", subcore_axis_name="s")
    n_chunks = n_groups * _G
    kfn = pl.kernel(
        functools.partial(_gather_body, n_groups, d),
        out_type=jax.ShapeDtypeStruct((_NW * n_chunks * _CHUNK, d), table.dtype),
        mesh=mesh,
        scratch_types=[
            pltpu.VMEM((n_chunks, _CHUNK), jnp.int32),
            pltpu.VMEM((_G * _CHUNK, d), table.dtype),
            pltpu.VMEM((_G * _CHUNK, d), table.dtype),
            pltpu.SemaphoreType.DMA,
            pltpu.SemaphoreType.DMA,
            pltpu.SemaphoreType.DMA,
            pltpu.SemaphoreType.DMA,
        ],
        compiler_params=pltpu.CompilerParams(use_tc_tiling_on_sc=False),
    )
    return kfn(table, idx)


def kernel(weight_tensor, index_tensor):
    b0, b1 = index_tensor.shape
    v, d = weight_tensor.shape
    n = b0 * b1
    per_group = _NW * _G * _CHUNK
    assert n % (2 * per_group) == 0
    n_groups = n // per_group
    pad = 128 // d
    wp = jnp.broadcast_to(weight_tensor[:, None, :], (v, pad, d)).reshape(
        v * pad, d)
    idx = index_tensor.astype(jnp.int32).reshape(_NW, n_groups * _G, _CHUNK)
    out = _gather(wp, idx * pad, n_groups, d)
    return out.reshape(b0, b1, d)


# speedup vs baseline: 3.5522x; 3.5522x over previous
import functools

import jax
import jax.numpy as jnp
from jax import lax
from jax.experimental import pallas as pl
from jax.experimental.pallas import tpu as pltpu
from jax.experimental.pallas import tpu_sc as plsc

_NC = 2
_NS = 16
_NW = _NC * _NS
_CHUNK = 512
_G = 1


def _gather_body(n_groups, d, table_hbm, idx_hbm, out_hbm,
                 idx_v, rows0, rows1, gsem0, gsem1, ssem0, ssem1):
    wid = lax.axis_index("s") * _NC + lax.axis_index("c")
    group_rows = _G * _CHUNK
    rows = (rows0, rows1)
    gsem = (gsem0, gsem1)
    ssem = (ssem0, ssem1)
    pltpu.sync_copy(idx_hbm.at[wid], idx_v)

    def pair(t, carry):
        for b in range(2):
            g = 2 * t + b
            @pl.when(t >= 1)
            def _():
                pltpu.make_async_copy(
                    rows[b], out_hbm.at[pl.ds(0, group_rows)], ssem[b]).wait()

            handles = [
                pltpu.async_copy(
                    table_hbm.at[idx_v.at[g * _G + u]],
                    rows[b].at[pl.ds(u * _CHUNK, _CHUNK)], gsem[b])
                for u in range(_G)
            ]
            for h in handles:
                h.wait()
            pltpu.async_copy(
                rows[b],
                out_hbm.at[pl.ds(g * _NW * group_rows + wid * group_rows,
                                 group_rows)],
                ssem[b])
        return carry

    lax.fori_loop(0, n_groups // 2, pair, 0)
    for b in range(2):
        pltpu.make_async_copy(
            rows[b], out_hbm.at[pl.ds(0, group_rows)], ssem[b]).wait()


@functools.partial(jax.jit, static_argnames=("n_groups", "d"))
def _gather(table, idx, n_groups, d):
    mesh = plsc.VectorSubcoreMesh(core_axis_name="c", subcore_axis_name="s")
    n_chunks = n_groups * _G
    kfn = pl.kernel(
        functools.partial(_gather_body, n_groups, d),
        out_type=jax.ShapeDtypeStruct((_NW * n_chunks * _CHUNK, d), table.dtype),
        mesh=mesh,
        scratch_types=[
            pltpu.VMEM((n_chunks, _CHUNK), jnp.int32),
            pltpu.VMEM((_G * _CHUNK, d), table.dtype),
            pltpu.VMEM((_G * _CHUNK, d), table.dtype),
            pltpu.SemaphoreType.DMA,
            pltpu.SemaphoreType.DMA,
            pltpu.SemaphoreType.DMA,
            pltpu.SemaphoreType.DMA,
        ],
        compiler_params=pltpu.CompilerParams(use_tc_tiling_on_sc=False),
    )
    return kfn(table, idx)


def kernel(weight_tensor, index_tensor):
    b0, b1 = index_tensor.shape
    v, d = weight_tensor.shape
    assert b0 == _NW * _CHUNK and _G == 1
    n_groups = b1
    pad = 128 // d
    wp = jnp.pad(weight_tensor, ((0, 0), (0, 128 - d))).reshape(v * pad, d)
    idx = (index_tensor.astype(jnp.int32)
           .reshape(_NW, _CHUNK, b1).transpose(0, 2, 1)) * pad
    out = _gather(wp, idx, n_groups, d)
    return out.reshape(b1, b0, d).transpose(1, 0, 2)


# speedup vs baseline: 3.5574x; 1.0015x over previous
import functools

import jax
import jax.numpy as jnp
from jax import lax
from jax.experimental import pallas as pl
from jax.experimental.pallas import tpu as pltpu
from jax.experimental.pallas import tpu_sc as plsc

_NC = 2
_NS = 16
_NW = _NC * _NS
_CHUNK = 512
_G = 1


def _gather_body(n_groups, d, table_hbm, idx_hbm, out_hbm,
                 idx_v, rows0, rows1, gsem0, gsem1, ssem0, ssem1):
    wid = lax.axis_index("s") * _NC + lax.axis_index("c")
    group_rows = _G * _CHUNK
    rows = (rows0, rows1)
    gsem = (gsem0, gsem1)
    ssem = (ssem0, ssem1)
    pltpu.sync_copy(idx_hbm.at[wid], idx_v)

    def pair(t, carry):
        for b in range(2):
            g = 2 * t + b
            @pl.when(t >= 1)
            def _():
                pltpu.make_async_copy(
                    rows[b],
                    out_hbm.at[0, pl.ds(0, group_rows), pl.ds(0, d)],
                    ssem[b]).wait()

            handles = [
                pltpu.async_copy(
                    table_hbm.at[idx_v.at[g * _G + u]],
                    rows[b].at[pl.ds(u * _CHUNK, _CHUNK)], gsem[b])
                for u in range(_G)
            ]
            for h in handles:
                h.wait()
            pltpu.async_copy(
                rows[b],
                out_hbm.at[g, pl.ds(wid * group_rows, group_rows),
                           pl.ds(0, d)],
                ssem[b])
        return carry

    lax.fori_loop(0, n_groups // 2, pair, 0)
    for b in range(2):
        pltpu.make_async_copy(
            rows[b], out_hbm.at[0, pl.ds(0, group_rows), pl.ds(0, d)],
            ssem[b]).wait()


@functools.partial(jax.jit, static_argnames=("n_groups", "d"))
def _gather(table, idx, n_groups, d):
    mesh = plsc.VectorSubcoreMesh(core_axis_name="c", subcore_axis_name="s")
    n_chunks = n_groups * _G
    kfn = pl.kernel(
        functools.partial(_gather_body, n_groups, d),
        out_type=jax.ShapeDtypeStruct(
            (n_groups, _NW * _G * _CHUNK, 128), table.dtype),
        mesh=mesh,
        scratch_types=[
            pltpu.VMEM((n_chunks, _CHUNK), jnp.int32),
            pltpu.VMEM((_G * _CHUNK, d), table.dtype),
            pltpu.VMEM((_G * _CHUNK, d), table.dtype),
            pltpu.SemaphoreType.DMA,
            pltpu.SemaphoreType.DMA,
            pltpu.SemaphoreType.DMA,
            pltpu.SemaphoreType.DMA,
        ],
        compiler_params=pltpu.CompilerParams(use_tc_tiling_on_sc=False),
    )
    return kfn(table, idx)


def kernel(weight_tensor, index_tensor):
    b0, b1 = index_tensor.shape
    v, d = weight_tensor.shape
    assert b0 == _NW * _CHUNK and _G == 1
    n_groups = b1
    pad = 128 // d
    wp = jnp.pad(weight_tensor, ((0, 0), (0, 128 - d))).reshape(v * pad, d)
    idx = (index_tensor.astype(jnp.int32)
           .reshape(_NW, _CHUNK, b1).transpose(0, 2, 1)) * pad
    out = _gather(wp, idx, n_groups, d)
    return out[:, :, :d].transpose(1, 0, 2)


# speedup vs baseline: 3.6142x; 1.0160x over previous
import functools

import jax
import jax.numpy as jnp
from jax import lax
from jax.experimental import pallas as pl
from jax.experimental.pallas import tpu as pltpu
from jax.experimental.pallas import tpu_sc as plsc

_NC = 2
_NS = 16
_NW = _NC * _NS
_CHUNK = 512
_G = 1


def _gather_body(n_groups, d, table_hbm, idx_hbm, out_hbm,
                 idx_v, rows0, rows1, gsem0, gsem1, ssem0, ssem1):
    wid = lax.axis_index("s") * _NC + lax.axis_index("c")
    group_rows = _G * _CHUNK
    rows = (rows0, rows1)
    gsem = (gsem0, gsem1)
    ssem = (ssem0, ssem1)
    pltpu.sync_copy(idx_hbm.at[wid], idx_v)

    def _drain_store(b):
        pltpu.make_async_copy(
            rows[b], out_hbm.at[0, pl.ds(0, group_rows), pl.ds(0, d)],
            ssem[b]).wait()

    pltpu.async_copy(table_hbm.at[idx_v.at[0]], rows[0], gsem[0])

    def pair(t, carry):
        for b in range(2):
            g = 2 * t + b
            if b == 0:
                @pl.when(t == 0)
                def _():
                    pltpu.async_copy(
                        table_hbm.at[idx_v.at[1]], rows[1], gsem[1])
                @pl.when(t >= 1)
                def _():
                    _drain_store(1)
                    pltpu.async_copy(
                        table_hbm.at[idx_v.at[g + 1]], rows[1], gsem[1])
            else:
                @pl.when(t < n_groups // 2 - 1)
                def _():
                    _drain_store(0)
                    pltpu.async_copy(
                        table_hbm.at[idx_v.at[g + 1]], rows[0], gsem[0])
            pltpu.make_async_copy(
                table_hbm.at[idx_v.at[0]], rows[b], gsem[b]).wait()
            pltpu.async_copy(
                rows[b],
                out_hbm.at[g, pl.ds(wid * group_rows, group_rows),
                           pl.ds(0, d)],
                ssem[b])
        return carry

    lax.fori_loop(0, n_groups // 2, pair, 0)
    for b in range(2):
        pltpu.make_async_copy(
            rows[b], out_hbm.at[0, pl.ds(0, group_rows), pl.ds(0, d)],
            ssem[b]).wait()


@functools.partial(jax.jit, static_argnames=("n_groups", "d"))
def _gather(table, idx, n_groups, d):
    mesh = plsc.VectorSubcoreMesh(core_axis_name="c", subcore_axis_name="s")
    n_chunks = n_groups * _G
    kfn = pl.kernel(
        functools.partial(_gather_body, n_groups, d),
        out_type=jax.ShapeDtypeStruct(
            (n_groups, _NW * _G * _CHUNK, 128), table.dtype),
        mesh=mesh,
        scratch_types=[
            pltpu.VMEM((n_chunks, _CHUNK), jnp.int32),
            pltpu.VMEM((_G * _CHUNK, d), table.dtype),
            pltpu.VMEM((_G * _CHUNK, d), table.dtype),
            pltpu.SemaphoreType.DMA,
            pltpu.SemaphoreType.DMA,
            pltpu.SemaphoreType.DMA,
            pltpu.SemaphoreType.DMA,
        ],
        compiler_params=pltpu.CompilerParams(use_tc_tiling_on_sc=False),
    )
    return kfn(table, idx)


def kernel(weight_tensor, index_tensor):
    b0, b1 = index_tensor.shape
    v, d = weight_tensor.shape
    assert b0 == _NW * _CHUNK and _G == 1
    n_groups = b1
    pad = 128 // d
    wp = jnp.pad(weight_tensor, ((0, 0), (0, 128 - d))).reshape(v * pad, d)
    idx = (index_tensor.astype(jnp.int32)
           .reshape(_NW, _CHUNK, b1).transpose(0, 2, 1)) * pad
    out = _gather(wp, idx, n_groups, d)
    return out[:, :, :d].transpose(1, 0, 2)
